# baseline (speedup 1.0000x reference)
import functools

import jax
import jax.numpy as jnp
from jax import lax
from jax.experimental import pallas as pl
from jax.experimental.pallas import tpu as pltpu
from jax.experimental.pallas import tpu_sc as plsc

_NUM_CLASSES = 21
_BG = 20
_RATIO = 3
_N = 131072
_C = 8192

_NW_SC = 32
_CHUNK = _N // _NW_SC
_L = 16
_UNROLL = 8


def _sc_count_body(gt_hbm, cnt_hbm, gt_v, cnt_v):
    wid = lax.axis_index("s") * 2 + lax.axis_index("c")
    pltpu.sync_copy(gt_hbm.at[pl.ds(wid * _CHUNK, _CHUNK)], gt_v)

    def body(i, acc):
        for u in range(_UNROLL):
            v = gt_v[pl.ds((i * _UNROLL + u) * _L, _L)]
            acc = acc + jnp.where(v == _BG, 1, 0)
        return acc

    acc = lax.fori_loop(0, _CHUNK // (_L * _UNROLL), body,
                        jnp.zeros((_L,), jnp.int32))
    cnt_v[...] = acc
    pltpu.sync_copy(cnt_v, cnt_hbm.at[wid])


def _sc_mine_body(gt_hbm, cnt_hbm, nsp_hbm, out_hbm, gt_v, cnt_v, nsp_v,
                  acc_v):
    wid = lax.axis_index("s") * 2 + lax.axis_index("c")
    base = wid * _CHUNK
    pltpu.sync_copy(gt_hbm.at[pl.ds(base, _CHUNK)], gt_v)
    pltpu.sync_copy(nsp_hbm.at[pl.ds(base, _CHUNK)], nsp_v)
    pltpu.sync_copy(cnt_hbm, cnt_v)

    def cbody(w, carry):
        pref, tot = carry
        cw = jnp.sum(cnt_v[w])
        return (pref + jnp.where(w < wid, cw, 0), tot + cw)

    pref, tot = lax.fori_loop(0, _NW_SC, cbody,
                              (jnp.int32(0), jnp.int32(0)))
    k = _RATIO * (_N - tot)

    def sbody(i, carry):
        rank, acc = carry
        for u in range(_UNROLL):
            off = (i * _UNROLL + u) * _L
            v = gt_v[pl.ds(off, _L)]
            neg = v == _BG
            incl = jnp.cumsum(jnp.where(neg, 1, 0))
            r = rank + incl - 1
            sel = neg & (r < k)
            acc = acc + jnp.where(sel, nsp_v[pl.ds(off, _L)], 0.0)
            rank = rank + incl[_L - 1]
        return (rank, acc)

    _, acc = lax.fori_loop(0, _CHUNK // (_L * _UNROLL), sbody,
                           (pref, jnp.zeros((_L,), jnp.float32)))
    acc_v[...] = acc
    pltpu.sync_copy(acc_v, out_hbm.at[wid])


def _sc_count(gt):
    mesh = plsc.VectorSubcoreMesh(core_axis_name="c", subcore_axis_name="s")
    return pl.kernel(
        _sc_count_body,
        mesh=mesh,
        compiler_params=pltpu.CompilerParams(needs_layout_passes=False),
        out_type=jax.ShapeDtypeStruct((_NW_SC, _L), jnp.int32),
        scratch_types=[
            pltpu.VMEM((_CHUNK,), jnp.int32),
            pltpu.VMEM((_L,), jnp.int32),
        ],
    )(gt)


def _sc_mine(gt, cnt, nsp):
    mesh = plsc.VectorSubcoreMesh(core_axis_name="c", subcore_axis_name="s")
    return pl.kernel(
        _sc_mine_body,
        mesh=mesh,
        compiler_params=pltpu.CompilerParams(needs_layout_passes=False),
        out_type=jax.ShapeDtypeStruct((_NW_SC, _L), jnp.float32),
        scratch_types=[
            pltpu.VMEM((_CHUNK,), jnp.int32),
            pltpu.VMEM((_NW_SC, _L), jnp.int32),
            pltpu.VMEM((_CHUNK,), jnp.float32),
            pltpu.VMEM((_L,), jnp.float32),
        ],
    )(gt, cnt, nsp)


def _tc_body(cats_ref, bbs_ref, gtb_ref, gt_ref, out_ref, nsp_ref, acc_ref):
    j = pl.program_id(0)

    @pl.when(j == 0)
    def _init():
        acc_ref[0] = 0.0
        acc_ref[1] = 0.0
        acc_ref[2] = 0.0

    x = cats_ref[...]
    gt = gt_ref[...]
    posf = jnp.where(gt != _BG, 1.0, 0.0)

    sp = jnp.maximum(x, 0.0) + jnp.log1p(jnp.exp(-jnp.abs(x)))
    row = lax.broadcasted_iota(jnp.int32, x.shape, 0)
    colsum = jnp.sum(jnp.where(row < _BG, sp, 0.0), axis=0, keepdims=True)
    nsp_ref[...] = colsum - posf * colsum
    conf_part = jnp.sum(posf * colsum)
    xc_part = jnp.sum(jnp.where(row == gt, x, 0.0) * posf)

    d = bbs_ref[...] - gtb_ref[...]
    ad = jnp.abs(d)
    l1 = jnp.where(ad < 1.0, 0.5 * d * d, ad - 0.5)
    loc_part = jnp.sum(l1 * posf)
    np_part = jnp.sum(posf)

    acc_ref[0] += np_part
    acc_ref[1] += conf_part - xc_part
    acc_ref[2] += loc_part

    @pl.when(j == pl.num_programs(0) - 1)
    def _fini():
        out_ref[0] = acc_ref[0]
        out_ref[1] = acc_ref[1]
        out_ref[2] = acc_ref[2]


def _tc_loss(catsT, bbsT, gtbT, gt1):
    return pl.pallas_call(
        _tc_body,
        grid=(_N // _C,),
        in_specs=[
            pl.BlockSpec((_NUM_CLASSES, _C), lambda j: (0, j)),
            pl.BlockSpec((4, _C), lambda j: (0, j)),
            pl.BlockSpec((4, _C), lambda j: (0, j)),
            pl.BlockSpec((1, _C), lambda j: (0, j)),
        ],
        out_specs=[
            pl.BlockSpec(memory_space=pltpu.SMEM),
            pl.BlockSpec((1, _C), lambda j: (0, j)),
        ],
        out_shape=[
            jax.ShapeDtypeStruct((3,), jnp.float32),
            jax.ShapeDtypeStruct((1, _N), jnp.float32),
        ],
        scratch_shapes=[pltpu.SMEM((3,), jnp.float32)],
    )(catsT, bbsT, gtbT, gt1)


def kernel(bbs_preds, cats_preds, gt_bbs, gt_cats):
    gt = gt_cats.astype(jnp.int32)
    cnt = _sc_count(gt)
    scalars, nsp = _tc_loss(
        cats_preds.T,
        bbs_preds.T,
        gt_bbs.T,
        gt.reshape(1, _N),
    )
    parts = _sc_mine(gt, cnt, nsp.reshape(_N))
    n = scalars[0]
    conf = scalars[1] + jnp.sum(parts)
    loc = scalars[2]
    total = (conf + loc) / n
    return (total, loc, conf)


# speedup vs baseline: 1.0467x; 1.0467x over previous
import functools

import jax
import jax.numpy as jnp
from jax import lax
from jax.experimental import pallas as pl
from jax.experimental.pallas import tpu as pltpu
from jax.experimental.pallas import tpu_sc as plsc

_NUM_CLASSES = 21
_BG = 20
_RATIO = 3
_N = 131072
_C = 8192

_NW_SC = 32
_CHUNK = _N // _NW_SC
_L = 16
_UNROLL = 8


def _sc_count_body(gt_hbm, cnt_hbm, gt_v, cnt_v):
    wid = lax.axis_index("s") * 2 + lax.axis_index("c")
    pltpu.sync_copy(gt_hbm.at[pl.ds(wid * _CHUNK, _CHUNK)], gt_v)

    def body(i, acc):
        for u in range(_UNROLL):
            v = gt_v[pl.ds((i * _UNROLL + u) * _L, _L)]
            acc = acc + jnp.where(v == _BG, 1, 0)
        return acc

    acc = lax.fori_loop(0, _CHUNK // (_L * _UNROLL), body,
                        jnp.zeros((_L,), jnp.int32))
    cnt_v[...] = acc
    pltpu.sync_copy(cnt_v, cnt_hbm.at[wid])


def _sc_mine_body(gt_hbm, cnt_hbm, nsp_hbm, out_hbm, gt_v, cnt_v, nsp_v,
                  acc_v):
    wid = lax.axis_index("s") * 2 + lax.axis_index("c")
    base = wid * _CHUNK
    pltpu.sync_copy(gt_hbm.at[pl.ds(base, _CHUNK)], gt_v)
    pltpu.sync_copy(nsp_hbm.at[pl.ds(base, _CHUNK)], nsp_v)
    pltpu.sync_copy(cnt_hbm, cnt_v)

    def cbody(w, carry):
        pref, tot = carry
        cw = jnp.sum(cnt_v[w])
        return (pref + jnp.where(w < wid, cw, 0), tot + cw)

    pref, tot = lax.fori_loop(0, _NW_SC, cbody,
                              (jnp.int32(0), jnp.int32(0)))
    k = _RATIO * (_N - tot)

    def sbody(i, carry):
        rank, acc = carry
        for u in range(_UNROLL):
            off = (i * _UNROLL + u) * _L
            v = gt_v[pl.ds(off, _L)]
            neg = v == _BG
            incl = jnp.cumsum(jnp.where(neg, 1, 0))
            r = rank + incl - 1
            sel = neg & (r < k)
            acc = acc + jnp.where(sel, nsp_v[pl.ds(off, _L)], 0.0)
            rank = rank + incl[_L - 1]
        return (rank, acc)

    _, acc = lax.fori_loop(0, _CHUNK // (_L * _UNROLL), sbody,
                           (pref, jnp.zeros((_L,), jnp.float32)))
    acc_v[...] = acc
    pltpu.sync_copy(acc_v, out_hbm.at[wid])


def _sc_count(gt):
    mesh = plsc.VectorSubcoreMesh(core_axis_name="c", subcore_axis_name="s")
    return pl.kernel(
        _sc_count_body,
        mesh=mesh,
        compiler_params=pltpu.CompilerParams(needs_layout_passes=False, skip_device_barrier=True),
        out_type=jax.ShapeDtypeStruct((_NW_SC, _L), jnp.int32),
        scratch_types=[
            pltpu.VMEM((_CHUNK,), jnp.int32),
            pltpu.VMEM((_L,), jnp.int32),
        ],
    )(gt)


def _sc_mine(gt, cnt, nsp):
    mesh = plsc.VectorSubcoreMesh(core_axis_name="c", subcore_axis_name="s")
    return pl.kernel(
        _sc_mine_body,
        mesh=mesh,
        compiler_params=pltpu.CompilerParams(needs_layout_passes=False, skip_device_barrier=True),
        out_type=jax.ShapeDtypeStruct((_NW_SC, _L), jnp.float32),
        scratch_types=[
            pltpu.VMEM((_CHUNK,), jnp.int32),
            pltpu.VMEM((_NW_SC, _L), jnp.int32),
            pltpu.VMEM((_CHUNK,), jnp.float32),
            pltpu.VMEM((_L,), jnp.float32),
        ],
    )(gt, cnt, nsp)


def _tc_body(cats_ref, bbs_ref, gtb_ref, gt_ref, out_ref, nsp_ref,
             amain_ref, aloc_ref, anp_ref):
    j = pl.program_id(0)

    @pl.when(j == 0)
    def _init():
        amain_ref[...] = jnp.zeros_like(amain_ref)
        aloc_ref[...] = jnp.zeros_like(aloc_ref)
        anp_ref[...] = jnp.zeros_like(anp_ref)

    x = cats_ref[...]
    gt = gt_ref[...]
    posf = jnp.where(gt != _BG, 1.0, 0.0)

    sp = jnp.maximum(x, 0.0) + jnp.log1p(jnp.exp(-jnp.abs(x)))
    row = lax.broadcasted_iota(jnp.int32, x.shape, 0)
    colsum = jnp.sum(jnp.where(row < _BG, sp, 0.0), axis=0, keepdims=True)
    nsp_ref[...] = colsum - posf * colsum
    gtp = jnp.where(gt == _BG, _NUM_CLASSES + 2, gt)
    xc = jnp.sum(jnp.where(row == gtp, x, 0.0), axis=0, keepdims=True)

    d = bbs_ref[...] - gtb_ref[...]
    ad = jnp.abs(d)
    l1 = jnp.where(ad < 1.0, 0.5 * d * d, ad - 0.5)
    locs = jnp.sum(l1, axis=0, keepdims=True) * posf

    amain_ref[...] += posf * colsum - xc
    aloc_ref[...] += locs
    anp_ref[...] += posf

    @pl.when(j == pl.num_programs(0) - 1)
    def _fini():
        out_ref[0] = jnp.sum(anp_ref[...])
        out_ref[1] = jnp.sum(amain_ref[...])
        out_ref[2] = jnp.sum(aloc_ref[...])


def _tc_loss(catsT, bbsT, gtbT, gt1):
    return pl.pallas_call(
        _tc_body,
        grid=(_N // _C,),
        in_specs=[
            pl.BlockSpec((_NUM_CLASSES, _C), lambda j: (0, j)),
            pl.BlockSpec((4, _C), lambda j: (0, j)),
            pl.BlockSpec((4, _C), lambda j: (0, j)),
            pl.BlockSpec((1, _C), lambda j: (0, j)),
        ],
        out_specs=[
            pl.BlockSpec(memory_space=pltpu.SMEM),
            pl.BlockSpec((1, _C), lambda j: (0, j)),
        ],
        out_shape=[
            jax.ShapeDtypeStruct((3,), jnp.float32),
            jax.ShapeDtypeStruct((1, _N), jnp.float32),
        ],
        scratch_shapes=[
            pltpu.VMEM((1, _C), jnp.float32),
            pltpu.VMEM((1, _C), jnp.float32),
            pltpu.VMEM((1, _C), jnp.float32),
        ],
    )(catsT, bbsT, gtbT, gt1)


def kernel(bbs_preds, cats_preds, gt_bbs, gt_cats):
    gt = gt_cats.astype(jnp.int32)
    cnt = _sc_count(gt)
    scalars, nsp = _tc_loss(
        cats_preds.T,
        bbs_preds.T,
        gt_bbs.T,
        gt.reshape(1, _N),
    )
    parts = _sc_mine(gt, cnt, nsp.reshape(_N))
    n = scalars[0]
    conf = scalars[1] + jnp.sum(parts)
    loc = scalars[2]
    total = (conf + loc) / n
    return (total, loc, conf)
